# baseline (speedup 1.0000x reference)
import math

import jax
import jax.numpy as jnp
from jax import lax
from jax.experimental import pallas as pl
from jax.experimental.pallas import tpu as pltpu
from jax.experimental.pallas import tpu_sc as plsc

N_CLASSES = 32768
N_ROWS = 4096
SV = 0.1 / (N_CLASSES - 2)
CONF = 0.9
DELTA = CONF - SV
C_ROW = (N_CLASSES - 1) * SV * math.log(SV) + CONF * math.log(CONF)

K = 4
BRS = 32
G = N_ROWS // (K * BRS)


def _fused_body(tt_ref, pk_ref, x0, x1, x2, x3, o_ref):
    i = pl.program_id(0)
    blk = jnp.zeros((1, 1), jnp.float32)
    for k, xr in enumerate((x0, x1, x2, x3)):
        acc = xr[:, 0:128]
        for c in range(1, N_CLASSES // 128):
            acc = acc + xr[:, c * 128:(c + 1) * 128]
        rows = jnp.sum(acc, axis=1, keepdims=True)
        r0 = (k * G + i) * BRS
        tt = tt_ref[pl.ds(r0, BRS), :]
        pk = pk_ref[pl.ds(r0, BRS), :]
        mask = (tt != 0).astype(jnp.float32)
        blk = blk + jnp.sum(mask * (C_ROW - SV * rows - DELTA * pk))[None, None]

    @pl.when(i == 0)
    def _():
        o_ref[...] = blk

    @pl.when(i > 0)
    def _():
        o_ref[...] = o_ref[...] + blk


def _fused_loss(ty_true, picked, ty_prob):
    xspec = lambda k: pl.BlockSpec((BRS, N_CLASSES), lambda i, k=k: (k * G + i, 0))
    out = pl.pallas_call(
        _fused_body,
        grid=(G,),
        in_specs=[
            pl.BlockSpec((N_ROWS, 1), lambda i: (0, 0)),
            pl.BlockSpec((N_ROWS, 1), lambda i: (0, 0)),
            xspec(0), xspec(1), xspec(2), xspec(3),
        ],
        out_specs=pl.BlockSpec((1, 1), lambda i: (0, 0)),
        out_shape=jax.ShapeDtypeStruct((1, 1), jnp.float32),
    )(ty_true.reshape(N_ROWS, 1), picked.reshape(N_ROWS, 1),
      ty_prob, ty_prob, ty_prob, ty_prob)
    return out[0, 0]


_NC, _NS, _L = 2, 16, 16
_NW = _NC * _NS
_BPW = N_ROWS // _NW


def _sc_gather(flat_prob, ty_true):
    mesh = plsc.VectorSubcoreMesh(core_axis_name="c", subcore_axis_name="s")

    @pl.kernel(
        out_type=jax.ShapeDtypeStruct((N_ROWS,), jnp.float32),
        mesh=mesh,
        scratch_types=[
            pltpu.VMEM((_BPW,), jnp.int32),
            pltpu.VMEM((_BPW,), jnp.float32),
            pltpu.SemaphoreType.DMA,
        ],
    )
    def k(table_hbm, idx_hbm, out_hbm, idx_v, vals_v, sem):
        wid = lax.axis_index("s") * _NC + lax.axis_index("c")
        base = wid * _BPW
        pltpu.sync_copy(idx_hbm.at[pl.ds(base, _BPW)], idx_v)
        for c in range(_BPW // _L):
            sl = pl.ds(c * _L, _L)
            i_vec = (base + c * _L) + lax.iota(jnp.int32, _L)
            j_vec = idx_v[sl]
            idx_v[sl] = (
                (i_vec >> 3) * ((N_CLASSES // 128) * 1024)
                + (j_vec >> 7) * 1024
                + (i_vec & 7) * 128
                + (j_vec & 127)
            )
        pltpu.async_copy(table_hbm.at[idx_v], vals_v, sem).wait()
        pltpu.sync_copy(vals_v, out_hbm.at[pl.ds(base, _BPW)])

    return k(flat_prob, ty_true)


def kernel(ty_prob, ty_true):
    tiled_flat = (
        ty_prob.reshape(N_ROWS // 8, 8, N_CLASSES // 128, 128)
        .transpose(0, 2, 1, 3)
        .reshape(-1)
    )
    picked = _sc_gather(tiled_flat, ty_true)
    return _fused_loss(ty_true, picked, ty_prob)


# speedup vs baseline: 1.0386x; 1.0386x over previous
import math

import jax
import jax.numpy as jnp
from jax import lax
from jax.experimental import pallas as pl
from jax.experimental.pallas import tpu as pltpu
from jax.experimental.pallas import tpu_sc as plsc

N_CLASSES = 32768
N_ROWS = 4096
SV = 0.1 / (N_CLASSES - 2)
CONF = 0.9
DELTA = CONF - SV
C_ROW = (N_CLASSES - 1) * SV * math.log(SV) + CONF * math.log(CONF)

BR = 128


def _fused_body(tt_ref, x_ref, o_ref):
    i = pl.program_id(0)
    acc = x_ref[:, 0:128]
    for c in range(1, N_CLASSES // 128):
        acc = acc + x_ref[:, c * 128:(c + 1) * 128]
    rows = jnp.sum(acc, axis=1, keepdims=True)
    tt = tt_ref[pl.ds(i * BR, BR), :]
    mask = (tt != 0).astype(jnp.float32)
    blk = jnp.sum(mask * (C_ROW - SV * rows))

    @pl.when(i == 0)
    def _():
        o_ref[...] = blk[None, None]

    @pl.when(i > 0)
    def _():
        o_ref[...] = o_ref[...] + blk[None, None]


def _dense_part(ty_true, ty_prob):
    out = pl.pallas_call(
        _fused_body,
        grid=(N_ROWS // BR,),
        in_specs=[
            pl.BlockSpec((N_ROWS, 1), lambda i: (0, 0)),
            pl.BlockSpec((BR, N_CLASSES), lambda i: (i, 0)),
        ],
        out_specs=pl.BlockSpec((1, 1), lambda i: (0, 0)),
        out_shape=jax.ShapeDtypeStruct((1, 1), jnp.float32),
    )(ty_true.reshape(N_ROWS, 1), ty_prob)
    return out[0, 0]


_NC, _NS, _L = 2, 16, 16
_NW = _NC * _NS
_BPW = N_ROWS // _NW


def _sc_gather_sum(flat_prob, ty_true):
    mesh = plsc.VectorSubcoreMesh(core_axis_name="c", subcore_axis_name="s")

    @pl.kernel(
        out_type=jax.ShapeDtypeStruct((_NW, _L), jnp.float32),
        mesh=mesh,
        scratch_types=[
            pltpu.VMEM((_BPW,), jnp.int32),
            pltpu.VMEM((_BPW,), jnp.int32),
            pltpu.VMEM((_BPW,), jnp.float32),
            pltpu.VMEM((_L,), jnp.float32),
            pltpu.SemaphoreType.DMA,
        ],
    )
    def k(table_hbm, idx_hbm, out_hbm, idx_v, addr_v, vals_v, acc_v, sem):
        wid = lax.axis_index("s") * _NC + lax.axis_index("c")
        base = wid * _BPW
        pltpu.sync_copy(idx_hbm.at[pl.ds(base, _BPW)], idx_v)
        for c in range(_BPW // _L):
            sl = pl.ds(c * _L, _L)
            i_vec = (base + c * _L) + lax.iota(jnp.int32, _L)
            j_vec = idx_v[sl]
            addr_v[sl] = (
                (i_vec >> 3) * ((N_CLASSES // 128) * 1024)
                + (j_vec >> 7) * 1024
                + (i_vec & 7) * 128
                + (j_vec & 127)
            )
        pltpu.async_copy(table_hbm.at[addr_v], vals_v, sem).wait()
        acc = jnp.zeros((_L,), jnp.float32)
        for c in range(_BPW // _L):
            sl = pl.ds(c * _L, _L)
            acc = acc + jnp.where(idx_v[sl] != 0, vals_v[sl],
                                  jnp.zeros((_L,), jnp.float32))
        acc_v[...] = acc
        pltpu.sync_copy(acc_v, out_hbm.at[wid])

    return k(flat_prob, ty_true)


def kernel(ty_prob, ty_true):
    tiled_flat = (
        ty_prob.reshape(N_ROWS // 8, 8, N_CLASSES // 128, 128)
        .transpose(0, 2, 1, 3)
        .reshape(-1)
    )
    picked_partials = _sc_gather_sum(tiled_flat, ty_true)
    dense = _dense_part(ty_true, ty_prob)
    return dense - DELTA * jnp.sum(picked_partials)
